# baseline (speedup 1.0000x reference)
import jax
import jax.numpy as jnp
from jax import lax
from jax.experimental import pallas as pl
from jax.experimental.pallas import tpu as pltpu
from jax.experimental.pallas import tpu_sc as plsc

B = 16384
ND = 13
NS = 26
V = 100000
D = 16
NF = ND + NS
WPF = V * D

_info = plsc.get_sparse_core_info()
NC = _info.num_cores
NSUB = _info.num_subcores
L = _info.num_lanes
NW = NC * NSUB
ROWS_W = B // NW
CHUNK = 128
NCHUNK = ROWS_W // CHUNK


def _sc_pool_body(inputs_hbm, t1_hbm, fm_hbm, inp_v, idx_fm, ridx, rows,
                  sacc, sqacc, outT, sem):
    wid = lax.axis_index("s") * NC + lax.axis_index("c")
    base = wid * ROWS_W

    pltpu.sync_copy(inputs_hbm.at[pl.ds(base, ROWS_W)], inp_v)

    lanes = lax.broadcasted_iota(jnp.int32, (L,), 0)
    off_a = lanes * ROWS_W
    off_b = (lanes + (NS - L)) * ROWS_W

    def trans_body(r, carry):
        a = inp_v[r, pl.ds(ND, L)].astype(jnp.int32)
        b = inp_v[r, pl.ds(NF - L, L)].astype(jnp.int32)
        plsc.store_scatter(idx_fm, [off_a + r], a)
        plsc.store_scatter(idx_fm, [off_b + r], b)
        return carry

    lax.fori_loop(0, ROWS_W, trans_body, None)

    zero = jnp.zeros((L,), jnp.float32)
    NCF = NCHUNK * NS
    DB = D * CHUNK

    def cf_body(cf, carry):
        @pl.when(cf < NCF)
        def _fire():
            c = cf // NS
            f = cf % NS
            ib = f * ROWS_W + c * CHUNK
            half = (cf % 3) * DB
            for g in range(CHUNK // L):
                v = idx_fm[pl.ds(ib + g * L, L)]
                rb = v + f * WPF
                for d in range(D):
                    ridx[(cf % 3) * D + d, pl.ds(g * L, L)] = rb + d * V
            for d in range(D):
                pltpu.async_copy(
                    t1_hbm.at[ridx.at[(cf % 3) * D + d]],
                    rows.at[pl.ds(half + d * CHUNK, CHUNK)],
                    sem,
                )

        @pl.when(cf > 1)
        def _acc():
            pcf = cf - 2
            c = pcf // NS
            f = pcf % NS
            half = (pcf % 3) * DB
            for d in range(D):
                pltpu.make_async_copy(
                    t1_hbm.at[ridx.at[(pcf % 3) * D + d]],
                    rows.at[pl.ds(half + d * CHUNK, CHUNK)],
                    sem,
                ).wait()

            @pl.when(f == 0)
            def _reset():
                for d in range(D):
                    for g in range(CHUNK // L):
                        sacc[d, pl.ds(g * L, L)] = zero
                        sqacc[d, pl.ds(g * L, L)] = zero

            for d in range(D):
                for g in range(CHUNK // L):
                    e = rows[pl.ds(half + d * CHUNK + g * L, L)]
                    plsc.addupdate(sacc.at[d, pl.ds(g * L, L)], e)
                    plsc.addupdate(sqacc.at[d, pl.ds(g * L, L)], e * e)

            @pl.when(f == NS - 1)
            def _finalize():
                for d in range(D):
                    for g in range(CHUNK // L):
                        s = sacc[d, pl.ds(g * L, L)]
                        q = sqacc[d, pl.ds(g * L, L)]
                        outT[d, pl.ds(c * CHUNK + g * L, L)] = 0.5 * (s * s - q)

        return carry

    lax.fori_loop(0, NCF + 2, cf_body, None)

    pltpu.sync_copy(outT, fm_hbm.at[:, pl.ds(base, ROWS_W)])


_sc_pool = pl.kernel(
    _sc_pool_body,
    out_type=jax.ShapeDtypeStruct((D, B), jnp.float32),
    mesh=plsc.VectorSubcoreMesh(core_axis_name="c", subcore_axis_name="s"),
    scratch_types=[
        pltpu.VMEM((ROWS_W, NF), jnp.float32),
        pltpu.VMEM((NS * ROWS_W,), jnp.int32),
        pltpu.VMEM((3 * D, CHUNK), jnp.int32),
        pltpu.VMEM((3 * D * CHUNK,), jnp.float32),
        pltpu.VMEM((D, CHUNK), jnp.float32),
        pltpu.VMEM((D, CHUNK), jnp.float32),
        pltpu.VMEM((D, ROWS_W), jnp.float32),
        pltpu.SemaphoreType.DMA,
    ],
    compiler_params=pltpu.CompilerParams(use_tc_tiling_on_sc=False,
                                         needs_layout_passes=False),
)


BT = 1024


def _mlp_body(inp_ref, fmt_ref, gamma_ref, beta_ref, mean_ref, var_ref,
              w1_ref, b1_ref, w2_ref, b2_ref, w3_ref, b3_ref, wo_ref, bo_ref,
              out_ref):
    fm = fmt_ref[...].T
    x = jnp.concatenate([inp_ref[:, :ND], fm], axis=1)
    scale = gamma_ref[...] * lax.rsqrt(var_ref[...] + 1e-3)
    x = (x - mean_ref[...]) * scale + beta_ref[...]
    h = jnp.maximum(
        jnp.dot(x, w1_ref[...], preferred_element_type=jnp.float32) + b1_ref[...], 0.0)
    h = jnp.maximum(
        jnp.dot(h, w2_ref[...], preferred_element_type=jnp.float32) + b2_ref[...], 0.0)
    h = jnp.maximum(
        jnp.dot(h, w3_ref[...], preferred_element_type=jnp.float32) + b3_ref[...], 0.0)
    o = jnp.dot(h, wo_ref[...], preferred_element_type=jnp.float32) + bo_ref[...]
    out_ref[...] = jax.nn.sigmoid(o)


def _full(shape):
    return pl.BlockSpec(shape, lambda i: tuple(0 for _ in shape))


_mlp = pl.pallas_call(
    _mlp_body,
    grid=(B // BT,),
    in_specs=[
        pl.BlockSpec((BT, NF), lambda i: (i, 0)),
        pl.BlockSpec((D, BT), lambda i: (0, i)),
        _full((ND + D,)), _full((ND + D,)), _full((ND + D,)), _full((ND + D,)),
        _full((ND + D, 256)), _full((256,)),
        _full((256, 128)), _full((128,)),
        _full((128, 64)), _full((64,)),
        _full((64, 1)), _full((1,)),
    ],
    out_specs=pl.BlockSpec((BT, 1), lambda i: (i, 0)),
    out_shape=jax.ShapeDtypeStruct((B, 1), jnp.float32),
)


def kernel(inputs, tables, gamma, beta, moving_mean, moving_var,
           W1, b1, W2, b2, W3, b3, Wo, bo):
    t1 = jnp.transpose(tables, (0, 2, 1)).reshape(NS * D * V)
    fmT = _sc_pool(inputs, t1)
    return _mlp(inputs, fmT, gamma, beta, moving_mean, moving_var,
                W1, b1, W2, b2, W3, b3, Wo, bo)


# speedup vs baseline: 1.0079x; 1.0079x over previous
import jax
import jax.numpy as jnp
from jax import lax
from jax.experimental import pallas as pl
from jax.experimental.pallas import tpu as pltpu
from jax.experimental.pallas import tpu_sc as plsc

B = 16384
ND = 13
NS = 26
V = 100000
D = 16
NF = ND + NS
WPF = V * D

_info = plsc.get_sparse_core_info()
NC = _info.num_cores
NSUB = _info.num_subcores
L = _info.num_lanes
NW = NC * NSUB
ROWS_W = B // NW
CHUNK = 128
NCHUNK = ROWS_W // CHUNK


def _sc_pool_body(inputs_hbm, t1_hbm, fm_hbm, inp_v, idx_fm, rows,
                  sacc, sqacc, outT, sem):
    wid = lax.axis_index("s") * NC + lax.axis_index("c")
    base = wid * ROWS_W

    pltpu.sync_copy(inputs_hbm.at[pl.ds(base, ROWS_W)], inp_v)

    lanes = lax.broadcasted_iota(jnp.int32, (L,), 0)
    off_a = lanes * ROWS_W
    off_b = (lanes + (NS - L)) * ROWS_W

    def trans_body(r, carry):
        a = inp_v[r, pl.ds(ND, L)].astype(jnp.int32)
        b = inp_v[r, pl.ds(NF - L, L)].astype(jnp.int32)
        plsc.store_scatter(idx_fm, [off_a + r], a)
        plsc.store_scatter(idx_fm, [off_b + r], b)
        return carry

    lax.fori_loop(0, ROWS_W, trans_body, None)

    zero = jnp.zeros((L,), jnp.float32)
    NCF = NCHUNK * NS
    DB = D * CHUNK

    def cf_body(cf, carry):
        @pl.when(cf < NCF)
        def _fire():
            c = cf // NS
            f = cf % NS
            ib = f * ROWS_W + c * CHUNK
            half = (cf % 3) * DB
            for d in range(D):
                pltpu.async_copy(
                    t1_hbm.at[f * D + d].at[idx_fm.at[pl.ds(ib, CHUNK)]],
                    rows.at[pl.ds(half + d * CHUNK, CHUNK)],
                    sem,
                )

        @pl.when(cf > 1)
        def _acc():
            pcf = cf - 2
            c = pcf // NS
            f = pcf % NS
            ibp = f * ROWS_W + c * CHUNK
            half = (pcf % 3) * DB
            for d in range(D):
                pltpu.make_async_copy(
                    t1_hbm.at[f * D + d].at[idx_fm.at[pl.ds(ibp, CHUNK)]],
                    rows.at[pl.ds(half + d * CHUNK, CHUNK)],
                    sem,
                ).wait()

            @pl.when(f == 0)
            def _reset():
                for d in range(D):
                    for g in range(CHUNK // L):
                        sacc[d, pl.ds(g * L, L)] = zero
                        sqacc[d, pl.ds(g * L, L)] = zero

            for d in range(D):
                for g in range(CHUNK // L):
                    e = rows[pl.ds(half + d * CHUNK + g * L, L)]
                    plsc.addupdate(sacc.at[d, pl.ds(g * L, L)], e)
                    plsc.addupdate(sqacc.at[d, pl.ds(g * L, L)], e * e)

            @pl.when(f == NS - 1)
            def _finalize():
                for d in range(D):
                    for g in range(CHUNK // L):
                        s = sacc[d, pl.ds(g * L, L)]
                        q = sqacc[d, pl.ds(g * L, L)]
                        outT[d, pl.ds(c * CHUNK + g * L, L)] = 0.5 * (s * s - q)

        return carry

    lax.fori_loop(0, NCF + 2, cf_body, None)

    pltpu.sync_copy(outT, fm_hbm.at[:, pl.ds(base, ROWS_W)])


_sc_pool = pl.kernel(
    _sc_pool_body,
    out_type=jax.ShapeDtypeStruct((D, B), jnp.float32),
    mesh=plsc.VectorSubcoreMesh(core_axis_name="c", subcore_axis_name="s"),
    scratch_types=[
        pltpu.VMEM((ROWS_W, NF), jnp.float32),
        pltpu.VMEM((NS * ROWS_W,), jnp.int32),
        pltpu.VMEM((3 * D * CHUNK,), jnp.float32),
        pltpu.VMEM((D, CHUNK), jnp.float32),
        pltpu.VMEM((D, CHUNK), jnp.float32),
        pltpu.VMEM((D, ROWS_W), jnp.float32),
        pltpu.SemaphoreType.DMA,
    ],
    compiler_params=pltpu.CompilerParams(use_tc_tiling_on_sc=False,
                                         needs_layout_passes=False),
)


BT = 1024


def _mlp_body(inp_ref, fmt_ref, gamma_ref, beta_ref, mean_ref, var_ref,
              w1_ref, b1_ref, w2_ref, b2_ref, w3_ref, b3_ref, wo_ref, bo_ref,
              out_ref):
    fm = fmt_ref[...].T
    x = jnp.concatenate([inp_ref[:, :ND], fm], axis=1)
    scale = gamma_ref[...] * lax.rsqrt(var_ref[...] + 1e-3)
    x = (x - mean_ref[...]) * scale + beta_ref[...]
    h = jnp.maximum(
        jnp.dot(x, w1_ref[...], preferred_element_type=jnp.float32) + b1_ref[...], 0.0)
    h = jnp.maximum(
        jnp.dot(h, w2_ref[...], preferred_element_type=jnp.float32) + b2_ref[...], 0.0)
    h = jnp.maximum(
        jnp.dot(h, w3_ref[...], preferred_element_type=jnp.float32) + b3_ref[...], 0.0)
    o = jnp.dot(h, wo_ref[...], preferred_element_type=jnp.float32) + bo_ref[...]
    out_ref[...] = jax.nn.sigmoid(o)


def _full(shape):
    return pl.BlockSpec(shape, lambda i: tuple(0 for _ in shape))


_mlp = pl.pallas_call(
    _mlp_body,
    grid=(B // BT,),
    in_specs=[
        pl.BlockSpec((BT, NF), lambda i: (i, 0)),
        pl.BlockSpec((D, BT), lambda i: (0, i)),
        _full((ND + D,)), _full((ND + D,)), _full((ND + D,)), _full((ND + D,)),
        _full((ND + D, 256)), _full((256,)),
        _full((256, 128)), _full((128,)),
        _full((128, 64)), _full((64,)),
        _full((64, 1)), _full((1,)),
    ],
    out_specs=pl.BlockSpec((BT, 1), lambda i: (i, 0)),
    out_shape=jax.ShapeDtypeStruct((B, 1), jnp.float32),
)


def kernel(inputs, tables, gamma, beta, moving_mean, moving_var,
           W1, b1, W2, b2, W3, b3, Wo, bo):
    t1 = jnp.transpose(tables, (0, 2, 1)).reshape(NS * D, V)
    fmT = _sc_pool(inputs, t1)
    return _mlp(inputs, fmT, gamma, beta, moving_mean, moving_var,
                W1, b1, W2, b2, W3, b3, Wo, bo)
